# Initial kernel scaffold; baseline (speedup 1.0000x reference)
import jax
import jax.numpy as jnp
from jax.experimental import pallas as pl


def kernel(x, params):
    raise NotImplementedError("write your pallas kernel here")


# speedup vs baseline: 2.6921x; 2.6921x over previous
import functools

import jax
import jax.numpy as jnp
from jax.experimental import pallas as pl
from jax.experimental.pallas import tpu as pltpu

B, T, C = 64, 128, 51
GD, GH, LD = 32, 64, 32
ROWS = B * C
TCH = 4
NG = T // TCH

_dot = functools.partial(jnp.dot, preferred_element_type=jnp.float32)


def _softmax(a):
    m = jnp.max(a, axis=-1, keepdims=True)
    e = jnp.exp(a - m)
    return e / jnp.sum(e, axis=-1, keepdims=True)


TF = 32


def _frontend_body(x_ref, a1_ref, a2_ref, w1_ref, b1_ref, w2_ref, b2_ref,
                   g_ref, be_ref, o_ref):
    xg = x_ref[...]
    xr = xg.reshape(8 * TF, C)
    a1 = _softmax(a1_ref[...])
    x1 = _dot(xr, a1.T)
    w1 = w1_ref[0]
    h = jnp.maximum(x1[:, :, None] * w1[None, None, :] + b1_ref[0], 0.0)
    a2 = _softmax(a2_ref[...])
    h2 = jax.lax.dot_general(a2, h, (((1,), (1,)), ((), ())),
                             preferred_element_type=jnp.float32)
    h3 = jnp.maximum(_dot(h2.reshape(C * 8 * TF, GD), w2_ref[...])
                     .reshape(C, 8 * TF, GD) + b2_ref[0], 0.0)
    mean = jnp.mean(h3, axis=-1, keepdims=True)
    var = jnp.mean((h3 - mean) * (h3 - mean), axis=-1, keepdims=True)
    h4 = (h3 - mean) * jax.lax.rsqrt(var + 1e-5) * g_ref[0] + be_ref[0]
    out = h4.reshape(C, 8, TF, GD).transpose(1, 0, 2, 3)
    o_ref[...] = out.reshape(8 * C, TF * GD)


def _gru_step(x, h, wi_ref, wh_ref, bi_ref, bh_ref):
    gi = _dot(x, wi_ref[...]) + bi_ref[0]
    gh = _dot(h, wh_ref[...]) + bh_ref[0]
    r = jax.nn.sigmoid(gi[:, :GH] + gh[:, :GH])
    z = jax.nn.sigmoid(gi[:, GH:2 * GH] + gh[:, GH:2 * GH])
    n = jnp.tanh(gi[:, 2 * GH:] + r * gh[:, 2 * GH:])
    return (1.0 - z) * n + z * h


def _gru1_body(xf_ref, xb_ref,
               wif_ref, whf_ref, bif_ref, bhf_ref,
               wib_ref, whb_ref, bib_ref, bhb_ref,
               yf_ref, yb_ref, hf_ref, hb_ref):
    @pl.when(pl.program_id(0) == 0)
    def _():
        hf_ref[...] = jnp.zeros_like(hf_ref)
        hb_ref[...] = jnp.zeros_like(hb_ref)

    xf = xf_ref[...]
    xb = xb_ref[...]
    hf = hf_ref[...]
    hb = hb_ref[...]
    yfs = []
    ybs = []
    for k in range(TCH):
        hf = _gru_step(xf[:, k * GD:(k + 1) * GD], hf,
                       wif_ref, whf_ref, bif_ref, bhf_ref)
        yfs.append(hf)
        kk = TCH - 1 - k
        hb = _gru_step(xb[:, kk * GD:(kk + 1) * GD], hb,
                       wib_ref, whb_ref, bib_ref, bhb_ref)
        ybs.insert(0, hb)
    hf_ref[...] = hf
    hb_ref[...] = hb
    yf_ref[...] = jnp.concatenate(yfs, axis=1)
    yb_ref[...] = jnp.concatenate(ybs, axis=1)


def _gru2_body(yf_ref, yb_ref, wif_ref, wib_ref, wh_ref, bi_ref, bh_ref,
               o_ref, h_ref):
    @pl.when(pl.program_id(0) == 0)
    def _():
        h_ref[...] = jnp.zeros_like(h_ref)

    yf = yf_ref[...]
    yb = yb_ref[...]
    h = h_ref[...]
    for k in range(TCH):
        gi = (_dot(yf[:, k * GH:(k + 1) * GH], wif_ref[...])
              + _dot(yb[:, k * GH:(k + 1) * GH], wib_ref[...]) + bi_ref[0])
        gh = _dot(h, wh_ref[...]) + bh_ref[0]
        r = jax.nn.sigmoid(gi[:, :GH] + gh[:, :GH])
        z = jax.nn.sigmoid(gi[:, GH:2 * GH] + gh[:, GH:2 * GH])
        n = jnp.tanh(gi[:, 2 * GH:] + r * gh[:, 2 * GH:])
        h = (1.0 - z) * n + z * h
    h_ref[...] = h
    o_ref[...] = h


def _mid_body(h2l_ref, yfl_ref, ybl_ref,
              wif_ref, wib_ref, bi_ref, bh_ref,
              pw_ref, pb_ref, mw_ref, mb_ref, lw_ref, lb_ref,
              dw_ref, db_ref,
              mu_ref, lv_ref, d_ref):
    yf_l = yfl_ref[:, (TCH - 1) * GH:]
    yb_l = ybl_ref[:, (TCH - 1) * GH:]
    gi = (_dot(yf_l, wif_ref[...]) + _dot(yb_l, wib_ref[...]) + bi_ref[0])
    gh = bh_ref[0]
    r = jax.nn.sigmoid(gi[:, :GH] + gh[:GH])
    z = jax.nn.sigmoid(gi[:, GH:2 * GH] + gh[GH:2 * GH])
    n = jnp.tanh(gi[:, 2 * GH:] + r * gh[2 * GH:])
    b2 = (1.0 - z) * n
    last = jnp.concatenate([h2l_ref[...], b2], axis=1)
    lastm = jnp.mean(last.reshape(B, C, 2 * GH), axis=1)
    hp = jnp.maximum(_dot(lastm, pw_ref[...]) + pb_ref[0], 0.0)
    mu = _dot(hp, mw_ref[...]) + mb_ref[0]
    lv = _dot(hp, lw_ref[...]) + lb_ref[0]
    d = jnp.maximum(_dot(mu, dw_ref[...]) + db_ref[0], 0.0)
    mu_ref[...] = mu
    lv_ref[...] = lv
    d_ref[...] = d


def _dec_body(d_ref, wi1_ref, wh1_ref, bi1_ref, bh1_ref,
              wi2_ref, wh2_ref, bi2_ref, bh2_ref, ow_ref, ob_ref, o_ref):
    gi1 = _dot(d_ref[...], wi1_ref[...]) + bi1_ref[0]
    zero = jnp.zeros((B, GH), jnp.float32)

    def lstm_step(g, c):
        i = jax.nn.sigmoid(g[:, :GH])
        f = jax.nn.sigmoid(g[:, GH:2 * GH])
        gg = jnp.tanh(g[:, 2 * GH:3 * GH])
        o = jax.nn.sigmoid(g[:, 3 * GH:])
        c = f * c + i * gg
        return jax.nn.sigmoid(g[:, 3 * GH:]) * jnp.tanh(c), c, o

    def body(t, carry):
        h1, c1, h2, c2 = carry
        g1 = gi1 + _dot(h1, wh1_ref[...]) + bh1_ref[0]
        h1, c1, _ = lstm_step(g1, c1)
        g2 = (_dot(h1, wi2_ref[...]) + bi2_ref[0]
              + _dot(h2, wh2_ref[...]) + bh2_ref[0])
        h2, c2, _ = lstm_step(g2, c2)
        xh = _dot(h2, ow_ref[...]) + ob_ref[0]
        o_ref[pl.ds(t, 1)] = xh[None]
        return (h1, c1, h2, c2)

    jax.lax.fori_loop(0, T, body, (zero, zero, zero, zero))


def _r2(v):
    return v.reshape(1, -1)


def kernel(x, params):
    p = params
    g = p['gru']
    seq_spec = pl.BlockSpec((ROWS, TCH * GD), lambda t: (0, t))
    y_spec = pl.BlockSpec((ROWS, TCH * GH), lambda t: (0, t))
    full = lambda shp: pl.BlockSpec(shp, lambda *_: tuple(0 for _ in shp))

    seq2d = pl.pallas_call(
        _frontend_body,
        grid=(B // 8, T // TF),
        in_specs=[pl.BlockSpec((8, TF, C), lambda i, j: (i, j, 0))] +
                 [full(s) for s in [(C, C), (C, C), (1, GD), (1, GD),
                                    (GD, GD), (1, GD), (1, GD), (1, GD)]],
        out_specs=pl.BlockSpec((8 * C, TF * GD), lambda i, j: (i, j)),
        out_shape=jax.ShapeDtypeStruct((ROWS, T * GD), jnp.float32),
    )(x, p['adj1'], p['adj2'], _r2(p['fc1_w'][:, 0]), _r2(p['fc1_b']),
      p['fc2_w'].T, _r2(p['fc2_b']), _r2(p['ln_g']), _r2(p['ln_b']))

    gf, gb = g[0][0], g[0][1]
    yf, yb = pl.pallas_call(
        _gru1_body,
        grid=(NG,),
        in_specs=[seq_spec,
                  pl.BlockSpec((ROWS, TCH * GD), lambda t: (0, NG - 1 - t))] +
                 [full(s) for s in [(GD, 3 * GH), (GH, 3 * GH),
                                    (1, 3 * GH), (1, 3 * GH)]] * 2,
        out_specs=[y_spec,
                   pl.BlockSpec((ROWS, TCH * GH), lambda t: (0, NG - 1 - t))],
        out_shape=[jax.ShapeDtypeStruct((ROWS, T * GH), jnp.float32)] * 2,
        scratch_shapes=[pltpu.VMEM((ROWS, GH), jnp.float32)] * 2,
    )(seq2d, seq2d,
      gf['wih'].T, gf['whh'].T, _r2(gf['bih']), _r2(gf['bhh']),
      gb['wih'].T, gb['whh'].T, _r2(gb['bih']), _r2(gb['bhh']))

    g2f, g2b = g[1][0], g[1][1]
    h2l = pl.pallas_call(
        _gru2_body,
        grid=(NG,),
        in_specs=[y_spec, y_spec] +
                 [full(s) for s in [(GH, 3 * GH), (GH, 3 * GH), (GH, 3 * GH),
                                    (1, 3 * GH), (1, 3 * GH)]],
        out_specs=pl.BlockSpec((ROWS, GH), lambda t: (0, 0)),
        out_shape=jax.ShapeDtypeStruct((ROWS, GH), jnp.float32),
        scratch_shapes=[pltpu.VMEM((ROWS, GH), jnp.float32)],
    )(yf, yb, g2f['wih'][:, :GH].T, g2f['wih'][:, GH:].T, g2f['whh'].T,
      _r2(g2f['bih']), _r2(g2f['bhh']))

    last_spec = pl.BlockSpec((ROWS, TCH * GH), lambda i: (0, NG - 1))
    mu, logv, dvec = pl.pallas_call(
        _mid_body,
        grid=(1,),
        in_specs=[full((ROWS, GH)), last_spec, last_spec] +
                 [full(s) for s in [(GH, 3 * GH), (GH, 3 * GH),
                                    (1, 3 * GH), (1, 3 * GH),
                                    (2 * GH, LD), (1, LD), (LD, LD), (1, LD),
                                    (LD, LD), (1, LD), (LD, GH), (1, GH)]],
        out_specs=[full((B, LD)), full((B, LD)), full((B, GH))],
        out_shape=[jax.ShapeDtypeStruct((B, LD), jnp.float32),
                   jax.ShapeDtypeStruct((B, LD), jnp.float32),
                   jax.ShapeDtypeStruct((B, GH), jnp.float32)],
    )(h2l, yf, yb,
      g2b['wih'][:, :GH].T, g2b['wih'][:, GH:].T,
      _r2(g2b['bih']), _r2(g2b['bhh']),
      p['pool_w'].T, _r2(p['pool_b']), p['mu_w'].T, _r2(p['mu_b']),
      p['lv_w'].T, _r2(p['lv_b']), p['dfc_w'].T, _r2(p['dfc_b']))

    l1, l2 = p['lstm']
    xh_tm = pl.pallas_call(
        _dec_body,
        in_specs=[full(s) for s in [(B, GH),
                                    (GH, 4 * GH), (GH, 4 * GH),
                                    (1, 4 * GH), (1, 4 * GH),
                                    (GH, 4 * GH), (GH, 4 * GH),
                                    (1, 4 * GH), (1, 4 * GH),
                                    (GH, C), (1, C)]],
        out_specs=full((T, B, C)),
        out_shape=jax.ShapeDtypeStruct((T, B, C), jnp.float32),
    )(dvec, l1['wih'].T, l1['whh'].T, _r2(l1['bih']), _r2(l1['bhh']),
      l2['wih'].T, l2['whh'].T, _r2(l2['bih']), _r2(l2['bhh']),
      p['out_w'].T, _r2(p['out_b']))

    x_hat = jnp.transpose(xh_tm, (1, 0, 2))
    return (x_hat, mu, logv)


# speedup vs baseline: 2.7204x; 1.0105x over previous
import functools

import jax
import jax.numpy as jnp
from jax.experimental import pallas as pl
from jax.experimental.pallas import tpu as pltpu

B, T, C = 64, 128, 51
GD, GH, LD = 32, 64, 32
ROWS = B * C
TCH = 4
NG = T // TCH

_dot = functools.partial(jnp.dot, preferred_element_type=jnp.float32)


def _softmax(a):
    m = jnp.max(a, axis=-1, keepdims=True)
    e = jnp.exp(a - m)
    return e / jnp.sum(e, axis=-1, keepdims=True)


TF = 8


def _frontend_body(x_ref, a1_ref, a2_ref, w1_ref, b1_ref, w2_ref, b2_ref,
                   g_ref, be_ref, o_ref):
    xg = x_ref[...]
    xr = xg.reshape(B * TF, C)
    a1 = _softmax(a1_ref[...])
    x1 = _dot(a1, xr.T)
    w1 = w1_ref[0]
    h = jnp.maximum(x1[:, :, None] * w1[None, None, :] + b1_ref[0], 0.0)
    a2 = _softmax(a2_ref[...])
    h2 = jax.lax.dot_general(a2, h, (((1,), (0,)), ((), ())),
                             preferred_element_type=jnp.float32)
    h3 = jnp.maximum(_dot(h2.reshape(C * B * TF, GD), w2_ref[...])
                     .reshape(C, B * TF, GD) + b2_ref[0], 0.0)
    mean = jnp.mean(h3, axis=-1, keepdims=True)
    var = jnp.mean((h3 - mean) * (h3 - mean), axis=-1, keepdims=True)
    h4 = (h3 - mean) * jax.lax.rsqrt(var + 1e-5) * g_ref[0] + be_ref[0]
    h5 = h4.reshape(C, B, TF, GD) + jnp.float32(0.0)
    o_ref[...] = h5.reshape(C * B, TF * GD)


def _gru_step(x, h, wi_ref, wh_ref, bi_ref, bh_ref):
    gi = _dot(x, wi_ref[...]) + bi_ref[0]
    gh = _dot(h, wh_ref[...]) + bh_ref[0]
    r = jax.nn.sigmoid(gi[:, :GH] + gh[:, :GH])
    z = jax.nn.sigmoid(gi[:, GH:2 * GH] + gh[:, GH:2 * GH])
    n = jnp.tanh(gi[:, 2 * GH:] + r * gh[:, 2 * GH:])
    return (1.0 - z) * n + z * h


def _gru1_body(xf_ref, xb_ref,
               wif_ref, whf_ref, bif_ref, bhf_ref,
               wib_ref, whb_ref, bib_ref, bhb_ref,
               yf_ref, yb_ref, hf_ref, hb_ref):
    @pl.when(pl.program_id(0) == 0)
    def _():
        hf_ref[...] = jnp.zeros_like(hf_ref)
        hb_ref[...] = jnp.zeros_like(hb_ref)

    xf = xf_ref[...]
    xb = xb_ref[...]
    hf = hf_ref[...]
    hb = hb_ref[...]
    yfs = []
    ybs = []
    for k in range(TCH):
        hf = _gru_step(xf[:, k * GD:(k + 1) * GD], hf,
                       wif_ref, whf_ref, bif_ref, bhf_ref)
        yfs.append(hf)
        kk = TCH - 1 - k
        hb = _gru_step(xb[:, kk * GD:(kk + 1) * GD], hb,
                       wib_ref, whb_ref, bib_ref, bhb_ref)
        ybs.insert(0, hb)
    hf_ref[...] = hf
    hb_ref[...] = hb
    yf_ref[...] = jnp.concatenate(yfs, axis=1)
    yb_ref[...] = jnp.concatenate(ybs, axis=1)


def _gru2_body(yf_ref, yb_ref, wif_ref, wib_ref, wh_ref, bi_ref, bh_ref,
               o_ref, h_ref):
    @pl.when(pl.program_id(0) == 0)
    def _():
        h_ref[...] = jnp.zeros_like(h_ref)

    yf = yf_ref[...]
    yb = yb_ref[...]
    h = h_ref[...]
    for k in range(TCH):
        gi = (_dot(yf[:, k * GH:(k + 1) * GH], wif_ref[...])
              + _dot(yb[:, k * GH:(k + 1) * GH], wib_ref[...]) + bi_ref[0])
        gh = _dot(h, wh_ref[...]) + bh_ref[0]
        r = jax.nn.sigmoid(gi[:, :GH] + gh[:, :GH])
        z = jax.nn.sigmoid(gi[:, GH:2 * GH] + gh[:, GH:2 * GH])
        n = jnp.tanh(gi[:, 2 * GH:] + r * gh[:, 2 * GH:])
        h = (1.0 - z) * n + z * h
    h_ref[...] = h
    o_ref[...] = h


def _mid_body(h2l_ref, yfl_ref, ybl_ref,
              wif_ref, wib_ref, bi_ref, bh_ref,
              pw_ref, pb_ref, mw_ref, mb_ref, lw_ref, lb_ref,
              dw_ref, db_ref,
              mu_ref, lv_ref, d_ref):
    yf_l = yfl_ref[:, (TCH - 1) * GH:]
    yb_l = ybl_ref[:, (TCH - 1) * GH:]
    gi = (_dot(yf_l, wif_ref[...]) + _dot(yb_l, wib_ref[...]) + bi_ref[0])
    gh = bh_ref[0]
    r = jax.nn.sigmoid(gi[:, :GH] + gh[:GH])
    z = jax.nn.sigmoid(gi[:, GH:2 * GH] + gh[GH:2 * GH])
    n = jnp.tanh(gi[:, 2 * GH:] + r * gh[2 * GH:])
    b2 = (1.0 - z) * n
    last = jnp.concatenate([h2l_ref[...], b2], axis=1)
    lastm = jnp.mean(last.reshape(C, B, 2 * GH), axis=0)
    hp = jnp.maximum(_dot(lastm, pw_ref[...]) + pb_ref[0], 0.0)
    mu = _dot(hp, mw_ref[...]) + mb_ref[0]
    lv = _dot(hp, lw_ref[...]) + lb_ref[0]
    d = jnp.maximum(_dot(mu, dw_ref[...]) + db_ref[0], 0.0)
    mu_ref[...] = mu
    lv_ref[...] = lv
    d_ref[...] = d


def _dec_body(d_ref, wi1_ref, wh1_ref, bi1_ref, bh1_ref,
              wi2_ref, wh2_ref, bi2_ref, bh2_ref, ow_ref, ob_ref, o_ref):
    gi1 = _dot(d_ref[...], wi1_ref[...]) + bi1_ref[0]
    zero = jnp.zeros((B, GH), jnp.float32)

    def lstm_step(g, c):
        i = jax.nn.sigmoid(g[:, :GH])
        f = jax.nn.sigmoid(g[:, GH:2 * GH])
        gg = jnp.tanh(g[:, 2 * GH:3 * GH])
        o = jax.nn.sigmoid(g[:, 3 * GH:])
        c = f * c + i * gg
        return jax.nn.sigmoid(g[:, 3 * GH:]) * jnp.tanh(c), c, o

    def body(t, carry):
        h1, c1, h2, c2 = carry
        g1 = gi1 + _dot(h1, wh1_ref[...]) + bh1_ref[0]
        h1, c1, _ = lstm_step(g1, c1)
        g2 = (_dot(h1, wi2_ref[...]) + bi2_ref[0]
              + _dot(h2, wh2_ref[...]) + bh2_ref[0])
        h2, c2, _ = lstm_step(g2, c2)
        xh = _dot(h2, ow_ref[...]) + ob_ref[0]
        o_ref[pl.ds(t, 1)] = xh[None]
        return (h1, c1, h2, c2)

    jax.lax.fori_loop(0, T, body, (zero, zero, zero, zero))


def _r2(v):
    return v.reshape(1, -1)


def kernel(x, params):
    p = params
    g = p['gru']
    seq_spec = pl.BlockSpec((ROWS, TCH * GD), lambda t: (0, t))
    y_spec = pl.BlockSpec((ROWS, TCH * GH), lambda t: (0, t))
    full = lambda shp: pl.BlockSpec(shp, lambda *_: tuple(0 for _ in shp))

    seq2d = pl.pallas_call(
        _frontend_body,
        grid=(T // TF,),
        in_specs=[pl.BlockSpec((B, TF, C), lambda j: (0, j, 0))] +
                 [full(s) for s in [(C, C), (C, C), (1, GD), (1, GD),
                                    (GD, GD), (1, GD), (1, GD), (1, GD)]],
        out_specs=pl.BlockSpec((C * B, TF * GD), lambda j: (0, j)),
        out_shape=jax.ShapeDtypeStruct((ROWS, T * GD), jnp.float32),
    )(x, p['adj1'], p['adj2'], _r2(p['fc1_w'][:, 0]), _r2(p['fc1_b']),
      p['fc2_w'].T, _r2(p['fc2_b']), _r2(p['ln_g']), _r2(p['ln_b']))

    gf, gb = g[0][0], g[0][1]
    yf, yb = pl.pallas_call(
        _gru1_body,
        grid=(NG,),
        in_specs=[seq_spec,
                  pl.BlockSpec((ROWS, TCH * GD), lambda t: (0, NG - 1 - t))] +
                 [full(s) for s in [(GD, 3 * GH), (GH, 3 * GH),
                                    (1, 3 * GH), (1, 3 * GH)]] * 2,
        out_specs=[y_spec,
                   pl.BlockSpec((ROWS, TCH * GH), lambda t: (0, NG - 1 - t))],
        out_shape=[jax.ShapeDtypeStruct((ROWS, T * GH), jnp.float32)] * 2,
        scratch_shapes=[pltpu.VMEM((ROWS, GH), jnp.float32)] * 2,
    )(seq2d, seq2d,
      gf['wih'].T, gf['whh'].T, _r2(gf['bih']), _r2(gf['bhh']),
      gb['wih'].T, gb['whh'].T, _r2(gb['bih']), _r2(gb['bhh']))

    g2f, g2b = g[1][0], g[1][1]
    h2l = pl.pallas_call(
        _gru2_body,
        grid=(NG,),
        in_specs=[y_spec, y_spec] +
                 [full(s) for s in [(GH, 3 * GH), (GH, 3 * GH), (GH, 3 * GH),
                                    (1, 3 * GH), (1, 3 * GH)]],
        out_specs=pl.BlockSpec((ROWS, GH), lambda t: (0, 0)),
        out_shape=jax.ShapeDtypeStruct((ROWS, GH), jnp.float32),
        scratch_shapes=[pltpu.VMEM((ROWS, GH), jnp.float32)],
    )(yf, yb, g2f['wih'][:, :GH].T, g2f['wih'][:, GH:].T, g2f['whh'].T,
      _r2(g2f['bih']), _r2(g2f['bhh']))

    last_spec = pl.BlockSpec((ROWS, TCH * GH), lambda i: (0, NG - 1))
    mu, logv, dvec = pl.pallas_call(
        _mid_body,
        grid=(1,),
        in_specs=[full((ROWS, GH)), last_spec, last_spec] +
                 [full(s) for s in [(GH, 3 * GH), (GH, 3 * GH),
                                    (1, 3 * GH), (1, 3 * GH),
                                    (2 * GH, LD), (1, LD), (LD, LD), (1, LD),
                                    (LD, LD), (1, LD), (LD, GH), (1, GH)]],
        out_specs=[full((B, LD)), full((B, LD)), full((B, GH))],
        out_shape=[jax.ShapeDtypeStruct((B, LD), jnp.float32),
                   jax.ShapeDtypeStruct((B, LD), jnp.float32),
                   jax.ShapeDtypeStruct((B, GH), jnp.float32)],
    )(h2l, yf, yb,
      g2b['wih'][:, :GH].T, g2b['wih'][:, GH:].T,
      _r2(g2b['bih']), _r2(g2b['bhh']),
      p['pool_w'].T, _r2(p['pool_b']), p['mu_w'].T, _r2(p['mu_b']),
      p['lv_w'].T, _r2(p['lv_b']), p['dfc_w'].T, _r2(p['dfc_b']))

    l1, l2 = p['lstm']
    xh_tm = pl.pallas_call(
        _dec_body,
        in_specs=[full(s) for s in [(B, GH),
                                    (GH, 4 * GH), (GH, 4 * GH),
                                    (1, 4 * GH), (1, 4 * GH),
                                    (GH, 4 * GH), (GH, 4 * GH),
                                    (1, 4 * GH), (1, 4 * GH),
                                    (GH, C), (1, C)]],
        out_specs=full((T, B, C)),
        out_shape=jax.ShapeDtypeStruct((T, B, C), jnp.float32),
    )(dvec, l1['wih'].T, l1['whh'].T, _r2(l1['bih']), _r2(l1['bhh']),
      l2['wih'].T, l2['whh'].T, _r2(l2['bih']), _r2(l2['bhh']),
      p['out_w'].T, _r2(p['out_b']))

    x_hat = jnp.transpose(xh_tm, (1, 0, 2))
    return (x_hat, mu, logv)


# speedup vs baseline: 2.9022x; 1.0668x over previous
import functools

import jax
import jax.numpy as jnp
from jax.experimental import pallas as pl
from jax.experimental.pallas import tpu as pltpu

B, T, C = 64, 128, 51
GD, GH, LD = 32, 64, 32
ROWS = B * C
TCH = 4
NG = T // TCH

_dot = functools.partial(jnp.dot, preferred_element_type=jnp.float32)


def _softmax(a):
    m = jnp.max(a, axis=-1, keepdims=True)
    e = jnp.exp(a - m)
    return e / jnp.sum(e, axis=-1, keepdims=True)


TF = 8


def _frontend_body(x_ref, a1_ref, a2_ref, w1_ref, b1_ref, w2_ref, b2_ref,
                   g_ref, be_ref, o_ref):
    xg = x_ref[...]
    xr = xg.reshape(B * TF, C)
    a1 = _softmax(a1_ref[...])
    x1 = _dot(a1, xr.T)
    w1 = w1_ref[0]
    h = jnp.maximum(x1[:, :, None] * w1[None, None, :] + b1_ref[0], 0.0)
    a2 = _softmax(a2_ref[...])
    h2 = jax.lax.dot_general(a2, h, (((1,), (0,)), ((), ())),
                             preferred_element_type=jnp.float32)
    h3 = jnp.maximum(_dot(h2.reshape(C * B * TF, GD), w2_ref[...])
                     .reshape(C, B * TF, GD) + b2_ref[0], 0.0)
    mean = jnp.mean(h3, axis=-1, keepdims=True)
    var = jnp.mean((h3 - mean) * (h3 - mean), axis=-1, keepdims=True)
    h4 = (h3 - mean) * jax.lax.rsqrt(var + 1e-5) * g_ref[0] + be_ref[0]
    h5 = h4.reshape(C, B, TF, GD) + jnp.float32(0.0)
    o_ref[...] = h5.reshape(C * B, TF * GD).astype(jnp.bfloat16)


def _gru_step(x, h, wi_ref, wh_ref, bi_ref, bh_ref):
    gi = _dot(x, wi_ref[...]) + bi_ref[0]
    gh = _dot(h, wh_ref[...]) + bh_ref[0]
    r = jax.nn.sigmoid(gi[:, :GH] + gh[:, :GH])
    z = jax.nn.sigmoid(gi[:, GH:2 * GH] + gh[:, GH:2 * GH])
    n = jnp.tanh(gi[:, 2 * GH:] + r * gh[:, 2 * GH:])
    return (1.0 - z) * n + z * h


def _gru1_body(xf_ref, xb_ref,
               wif_ref, whf_ref, bif_ref, bhf_ref,
               wib_ref, whb_ref, bib_ref, bhb_ref,
               yf_ref, yb_ref, hf_ref, hb_ref):
    @pl.when(pl.program_id(0) == 0)
    def _():
        hf_ref[...] = jnp.zeros_like(hf_ref)
        hb_ref[...] = jnp.zeros_like(hb_ref)

    xf = xf_ref[...].astype(jnp.float32)
    xb = xb_ref[...].astype(jnp.float32)
    hf = hf_ref[...]
    hb = hb_ref[...]
    yfs = []
    ybs = []
    for k in range(TCH):
        hf = _gru_step(xf[:, k * GD:(k + 1) * GD], hf,
                       wif_ref, whf_ref, bif_ref, bhf_ref)
        yfs.append(hf)
        kk = TCH - 1 - k
        hb = _gru_step(xb[:, kk * GD:(kk + 1) * GD], hb,
                       wib_ref, whb_ref, bib_ref, bhb_ref)
        ybs.insert(0, hb)
    hf_ref[...] = hf
    hb_ref[...] = hb
    yf_ref[...] = jnp.concatenate(yfs, axis=1).astype(jnp.bfloat16)
    yb_ref[...] = jnp.concatenate(ybs, axis=1).astype(jnp.bfloat16)


def _gru2_body(yf_ref, yb_ref, wif_ref, wib_ref, wh_ref, bi_ref, bh_ref,
               o_ref, h_ref):
    @pl.when(pl.program_id(0) == 0)
    def _():
        h_ref[...] = jnp.zeros_like(h_ref)

    yf = yf_ref[...].astype(jnp.float32)
    yb = yb_ref[...].astype(jnp.float32)
    h = h_ref[...]
    for k in range(TCH):
        gi = (_dot(yf[:, k * GH:(k + 1) * GH], wif_ref[...])
              + _dot(yb[:, k * GH:(k + 1) * GH], wib_ref[...]) + bi_ref[0])
        gh = _dot(h, wh_ref[...]) + bh_ref[0]
        r = jax.nn.sigmoid(gi[:, :GH] + gh[:, :GH])
        z = jax.nn.sigmoid(gi[:, GH:2 * GH] + gh[:, GH:2 * GH])
        n = jnp.tanh(gi[:, 2 * GH:] + r * gh[:, 2 * GH:])
        h = (1.0 - z) * n + z * h
    h_ref[...] = h
    o_ref[...] = h


def _mid_body(h2l_ref, yfl_ref, ybl_ref,
              wif_ref, wib_ref, bi_ref, bh_ref,
              pw_ref, pb_ref, mw_ref, mb_ref, lw_ref, lb_ref,
              dw_ref, db_ref,
              mu_ref, lv_ref, d_ref):
    yf_l = yfl_ref[:, (TCH - 1) * GH:].astype(jnp.float32)
    yb_l = ybl_ref[:, (TCH - 1) * GH:].astype(jnp.float32)
    gi = (_dot(yf_l, wif_ref[...]) + _dot(yb_l, wib_ref[...]) + bi_ref[0])
    gh = bh_ref[0]
    r = jax.nn.sigmoid(gi[:, :GH] + gh[:GH])
    z = jax.nn.sigmoid(gi[:, GH:2 * GH] + gh[GH:2 * GH])
    n = jnp.tanh(gi[:, 2 * GH:] + r * gh[2 * GH:])
    b2 = (1.0 - z) * n
    last = jnp.concatenate([h2l_ref[...], b2], axis=1)
    lastm = jnp.mean(last.reshape(C, B, 2 * GH), axis=0)
    hp = jnp.maximum(_dot(lastm, pw_ref[...]) + pb_ref[0], 0.0)
    mu = _dot(hp, mw_ref[...]) + mb_ref[0]
    lv = _dot(hp, lw_ref[...]) + lb_ref[0]
    d = jnp.maximum(_dot(mu, dw_ref[...]) + db_ref[0], 0.0)
    mu_ref[...] = mu
    lv_ref[...] = lv
    d_ref[...] = d


def _dec_body(d_ref, wi1_ref, wh1_ref, bi1_ref, bh1_ref,
              wi2_ref, wh2_ref, bi2_ref, bh2_ref, ow_ref, ob_ref, o_ref):
    gi1 = _dot(d_ref[...], wi1_ref[...]) + bi1_ref[0]
    zero = jnp.zeros((B, GH), jnp.float32)

    def lstm_step(g, c):
        i = jax.nn.sigmoid(g[:, :GH])
        f = jax.nn.sigmoid(g[:, GH:2 * GH])
        gg = jnp.tanh(g[:, 2 * GH:3 * GH])
        o = jax.nn.sigmoid(g[:, 3 * GH:])
        c = f * c + i * gg
        return jax.nn.sigmoid(g[:, 3 * GH:]) * jnp.tanh(c), c, o

    def body(t, carry):
        h1, c1, h2, c2 = carry
        g1 = gi1 + _dot(h1, wh1_ref[...]) + bh1_ref[0]
        h1, c1, _ = lstm_step(g1, c1)
        g2 = (_dot(h1, wi2_ref[...]) + bi2_ref[0]
              + _dot(h2, wh2_ref[...]) + bh2_ref[0])
        h2, c2, _ = lstm_step(g2, c2)
        xh = _dot(h2, ow_ref[...]) + ob_ref[0]
        o_ref[pl.ds(t, 1)] = xh[None]
        return (h1, c1, h2, c2)

    jax.lax.fori_loop(0, T, body, (zero, zero, zero, zero))


def _r2(v):
    return v.reshape(1, -1)


def kernel(x, params):
    p = params
    g = p['gru']
    seq_spec = pl.BlockSpec((ROWS, TCH * GD), lambda t: (0, t))
    y_spec = pl.BlockSpec((ROWS, TCH * GH), lambda t: (0, t))
    full = lambda shp: pl.BlockSpec(shp, lambda *_: tuple(0 for _ in shp))

    seq2d = pl.pallas_call(
        _frontend_body,
        grid=(T // TF,),
        in_specs=[pl.BlockSpec((B, TF, C), lambda j: (0, j, 0))] +
                 [full(s) for s in [(C, C), (C, C), (1, GD), (1, GD),
                                    (GD, GD), (1, GD), (1, GD), (1, GD)]],
        out_specs=pl.BlockSpec((C * B, TF * GD), lambda j: (0, j)),
        out_shape=jax.ShapeDtypeStruct((ROWS, T * GD), jnp.bfloat16),
    )(x, p['adj1'], p['adj2'], _r2(p['fc1_w'][:, 0]), _r2(p['fc1_b']),
      p['fc2_w'].T, _r2(p['fc2_b']), _r2(p['ln_g']), _r2(p['ln_b']))

    gf, gb = g[0][0], g[0][1]
    yf, yb = pl.pallas_call(
        _gru1_body,
        grid=(NG,),
        in_specs=[seq_spec,
                  pl.BlockSpec((ROWS, TCH * GD), lambda t: (0, NG - 1 - t))] +
                 [full(s) for s in [(GD, 3 * GH), (GH, 3 * GH),
                                    (1, 3 * GH), (1, 3 * GH)]] * 2,
        out_specs=[y_spec,
                   pl.BlockSpec((ROWS, TCH * GH), lambda t: (0, NG - 1 - t))],
        out_shape=[jax.ShapeDtypeStruct((ROWS, T * GH), jnp.bfloat16)] * 2,
        scratch_shapes=[pltpu.VMEM((ROWS, GH), jnp.float32)] * 2,
    )(seq2d, seq2d,
      gf['wih'].T, gf['whh'].T, _r2(gf['bih']), _r2(gf['bhh']),
      gb['wih'].T, gb['whh'].T, _r2(gb['bih']), _r2(gb['bhh']))

    g2f, g2b = g[1][0], g[1][1]
    h2l = pl.pallas_call(
        _gru2_body,
        grid=(NG,),
        in_specs=[y_spec, y_spec] +
                 [full(s) for s in [(GH, 3 * GH), (GH, 3 * GH), (GH, 3 * GH),
                                    (1, 3 * GH), (1, 3 * GH)]],
        out_specs=pl.BlockSpec((ROWS, GH), lambda t: (0, 0)),
        out_shape=jax.ShapeDtypeStruct((ROWS, GH), jnp.float32),
        scratch_shapes=[pltpu.VMEM((ROWS, GH), jnp.float32)],
    )(yf, yb, g2f['wih'][:, :GH].T, g2f['wih'][:, GH:].T, g2f['whh'].T,
      _r2(g2f['bih']), _r2(g2f['bhh']))

    last_spec = pl.BlockSpec((ROWS, TCH * GH), lambda i: (0, NG - 1))
    mu, logv, dvec = pl.pallas_call(
        _mid_body,
        grid=(1,),
        in_specs=[full((ROWS, GH)), last_spec, last_spec] +
                 [full(s) for s in [(GH, 3 * GH), (GH, 3 * GH),
                                    (1, 3 * GH), (1, 3 * GH),
                                    (2 * GH, LD), (1, LD), (LD, LD), (1, LD),
                                    (LD, LD), (1, LD), (LD, GH), (1, GH)]],
        out_specs=[full((B, LD)), full((B, LD)), full((B, GH))],
        out_shape=[jax.ShapeDtypeStruct((B, LD), jnp.float32),
                   jax.ShapeDtypeStruct((B, LD), jnp.float32),
                   jax.ShapeDtypeStruct((B, GH), jnp.float32)],
    )(h2l, yf, yb,
      g2b['wih'][:, :GH].T, g2b['wih'][:, GH:].T,
      _r2(g2b['bih']), _r2(g2b['bhh']),
      p['pool_w'].T, _r2(p['pool_b']), p['mu_w'].T, _r2(p['mu_b']),
      p['lv_w'].T, _r2(p['lv_b']), p['dfc_w'].T, _r2(p['dfc_b']))

    l1, l2 = p['lstm']
    xh_tm = pl.pallas_call(
        _dec_body,
        in_specs=[full(s) for s in [(B, GH),
                                    (GH, 4 * GH), (GH, 4 * GH),
                                    (1, 4 * GH), (1, 4 * GH),
                                    (GH, 4 * GH), (GH, 4 * GH),
                                    (1, 4 * GH), (1, 4 * GH),
                                    (GH, C), (1, C)]],
        out_specs=full((T, B, C)),
        out_shape=jax.ShapeDtypeStruct((T, B, C), jnp.float32),
    )(dvec, l1['wih'].T, l1['whh'].T, _r2(l1['bih']), _r2(l1['bhh']),
      l2['wih'].T, l2['whh'].T, _r2(l2['bih']), _r2(l2['bhh']),
      p['out_w'].T, _r2(p['out_b']))

    x_hat = jnp.transpose(xh_tm, (1, 0, 2))
    return (x_hat, mu, logv)
